# baseline (speedup 1.0000x reference)
import functools

import jax
import jax.numpy as jnp
from jax import lax
from jax.experimental import pallas as pl
from jax.experimental.pallas import tpu as pltpu
from jax.experimental.pallas import tpu_sc as plsc

_M = 256
_S = 16
_L = 8
_V = 1024
_N = _M * _S
_DLIT = 64
_DHID = 256
_NC = 2
_NS = 16
_NW = _NC * _NS
_NPW = _N // _NW
_SLOTS = _S * _L
_TROWS = _V + 1
_TSTRIDE = _DLIT + 1


def _sc_bagsum_body(table_hbm, ids_hbm, out_hbm, table_v, ids_v, stage_v):
    wid = lax.axis_index("s") * _NC + lax.axis_index("c")
    pltpu.sync_copy(table_hbm, table_v)
    pltpu.sync_copy(ids_hbm.at[wid], ids_v)
    cols = lax.iota(jnp.int32, 16)
    zero16 = jnp.zeros((16,), jnp.float32)

    def n_body(n, _):
        def sb_body(sb, accs):
            ids16 = ids_v[n, pl.ds(sb * 16, 16)]
            for j in range(16):
                idj = ids16.at[jnp.full((16,), j, jnp.int32)].get(
                    mode="promise_in_bounds")
                base = idj * _TSTRIDE + cols
                accs = tuple(
                    accs[k] + plsc.load_gather(table_v, [base + 16 * k])
                    for k in range(4)
                )
            return accs

        accs = lax.fori_loop(0, _SLOTS // 16, sb_body, (zero16,) * 4)
        for k in range(4):
            stage_v[pl.ds(n * _DLIT + k * 16, 16)] = accs[k]
        return 0

    lax.fori_loop(0, _NPW, n_body, 0)
    pltpu.sync_copy(stage_v, out_hbm.at[pl.ds(wid * _NPW * _DLIT, _NPW * _DLIT)])


def _sc_sel_body(nodes_hbm, idx_hbm, out_hbm, idx_v, rows_v, sem):
    wid = lax.axis_index("s") * _NC + lax.axis_index("c")
    base = wid * 32
    pltpu.sync_copy(idx_hbm.at[pl.ds(base, 32)], idx_v)
    pltpu.async_copy(nodes_hbm.at[idx_v], rows_v, sem).wait()
    pltpu.sync_copy(rows_v, out_hbm.at[pl.ds(base, 32)])


def _tc_body(calls_ref, bnode_ref, we_ref, ws0_ref, wm0_ref, ws1_ref, wm1_ref,
             out_ref, graph_ref):
    f32 = jnp.float32
    A3 = (calls_ref[...] > -1).astype(f32)
    deg2 = jnp.sum(A3, axis=2)
    col2 = jnp.sum(A3, axis=1)
    invdeg3 = (1.0 / jnp.maximum(deg2, 1.0))[..., None]

    node_sum = jnp.dot(bnode_ref[...], we_ref[...], preferred_element_type=f32)
    nodes3 = node_sum.reshape(_M, _S, 128) * invdeg3
    nodes = nodes3.reshape(_N, 128)

    def abmm(h3, d):
        acc = jnp.zeros((_M, _S, d), f32)
        for j in range(_S):
            acc = acc + A3[:, :, j:j + 1] * h3[:, j:j + 1, :]
        return (acc * invdeg3).reshape(_N, d)

    wm0 = wm0_ref[...]
    w0 = ws0_ref[...] + wm0[128:]
    h1 = jnp.maximum(
        jnp.dot(nodes, w0, preferred_element_type=f32)
        + jnp.dot(abmm(nodes3, 128), wm0[:128], preferred_element_type=f32),
        0.0)
    wm1 = wm1_ref[...]
    pre = (jnp.dot(h1, ws1_ref[...], preferred_element_type=f32)
           + jnp.dot(nodes, wm1[256:], preferred_element_type=f32)
           + jnp.dot(abmm(h1.reshape(_M, _S, _DHID), _DHID), wm1[:256],
                     preferred_element_type=f32))
    out = jnp.maximum(pre, 0.0)
    out_ref[...] = out
    mask3 = ((deg2 + col2) > 0.0).astype(f32)[..., None]
    msum = jnp.sum(jnp.sum(out.reshape(_M, _S, _DHID) * mask3, axis=1),
                   axis=0, keepdims=True)
    graph_ref[...] = msum / jnp.sum(mask3)


def kernel(formulas, calls, num_literals, rm_id, state_id, lit_table,
           W_edge, W_self0, W_msg0, W_self1, W_msg1):
    calls3 = calls[..., 0]
    a_bool = calls3 > -1
    nlit = jnp.maximum(num_literals[..., 0], 1)
    lmask = (jnp.arange(_L, dtype=jnp.int32)[None, None, None, :]
             < nlit[..., None]) & a_bool[..., None]
    ids = jnp.where(lmask, formulas[:, :, :, 0, :], _V).astype(jnp.int32)
    ids_t = ids.reshape(_NW, _NPW, _SLOTS)
    table_pad = jnp.pad(
        lit_table, ((0, 1), (0, _TSTRIDE - _DLIT))).reshape(-1)

    mesh = plsc.VectorSubcoreMesh(core_axis_name="c", subcore_axis_name="s",
                                  num_cores=_NC)

    bagsum = functools.partial(
        pl.kernel, mesh=mesh,
        compiler_params=pltpu.CompilerParams(needs_layout_passes=False),
        out_type=jax.ShapeDtypeStruct((_N * _DLIT,), jnp.float32),
        scratch_types=[
            pltpu.VMEM((_TROWS * _TSTRIDE,), jnp.float32),
            pltpu.VMEM((_NPW, _SLOTS), jnp.int32),
            pltpu.VMEM((_NPW * _DLIT,), jnp.float32),
        ],
    )(_sc_bagsum_body)
    bnode = bagsum(table_pad, ids_t).reshape(_N, _DLIT)

    out_nodes, graph = pl.pallas_call(
        _tc_body,
        out_shape=(jax.ShapeDtypeStruct((_N, _DHID), jnp.float32),
                   jax.ShapeDtypeStruct((1, _DHID), jnp.float32)),
    )(calls3, bnode, W_edge, W_self0, W_msg0, W_self1, W_msg1)

    idx_b = (rm_id * _S + state_id).astype(jnp.int32)
    sel = functools.partial(
        pl.kernel, mesh=mesh,
        out_type=jax.ShapeDtypeStruct((idx_b.shape[0], _DHID), jnp.float32),
        scratch_types=[
            pltpu.VMEM((32,), jnp.int32),
            pltpu.VMEM((32, _DHID), jnp.float32),
            pltpu.SemaphoreType.DMA,
        ],
    )(_sc_sel_body)(out_nodes, idx_b)

    left = jnp.broadcast_to(graph, (idx_b.shape[0], _DHID))
    return jnp.concatenate([left, sel], axis=1)


# speedup vs baseline: 1.1433x; 1.1433x over previous
import functools

import jax
import jax.numpy as jnp
from jax import lax
from jax.experimental import pallas as pl
from jax.experimental.pallas import tpu as pltpu
from jax.experimental.pallas import tpu_sc as plsc

_M = 256
_S = 16
_L = 8
_V = 1024
_N = _M * _S
_DLIT = 64
_DHID = 256
_NC = 2
_NS = 16
_NW = _NC * _NS
_NPW = _N // _NW
_SLOTS = _S * _L
_TROWS = _V + 1
_TW = _DLIT // 2
_TSTRIDE = _TW + 1


def _sc_bagsum_body(table_hbm, ids_hbm, out_hbm, table_v, ids_v, stage_v):
    wid = lax.axis_index("s") * _NC + lax.axis_index("c")
    pltpu.sync_copy(table_hbm, table_v)
    pltpu.sync_copy(ids_hbm.at[wid], ids_v)
    cols = lax.iota(jnp.int32, 16)
    zero16 = jnp.zeros((16,), jnp.float32)
    zbf = jnp.zeros((32,), jnp.bfloat16)

    def n_body(n, _):
        def sb_body(sb, faccs):
            ids16 = ids_v[n, pl.ds(sb * 16, 16)]
            p0, p1 = zbf, zbf
            for j in range(16):
                idj = ids16.at[jnp.full((16,), j, jnp.int32)].get(
                    mode="promise_in_bounds")
                base = idj * _TSTRIDE + cols
                g0 = plsc.load_gather(table_v, [base])
                g1 = plsc.load_gather(table_v, [base + 16])
                p0 = p0 + plsc.bitcast(g0, jnp.bfloat16)
                p1 = p1 + plsc.bitcast(g1, jnp.bfloat16)
            a0, b0 = plsc.unpack(p0, format=plsc.PackFormat.INTERLEAVED)
            a1, b1 = plsc.unpack(p1, format=plsc.PackFormat.INTERLEAVED)
            return (faccs[0] + a0, faccs[1] + b0, faccs[2] + a1, faccs[3] + b1)

        faccs = lax.fori_loop(0, _SLOTS // 16, sb_body, (zero16,) * 4)
        for k in range(4):
            off = (k // 2) * 32 + (k % 2)
            plsc.store_scatter(stage_v, [n * _DLIT + off + 2 * cols], faccs[k])
        return 0

    lax.fori_loop(0, _NPW, n_body, 0)
    pltpu.sync_copy(stage_v, out_hbm.at[pl.ds(wid * _NPW * _DLIT, _NPW * _DLIT)])


def _sc_sel_body(nodes_hbm, idx_hbm, out_hbm, idx_v, rows_v, sem):
    wid = lax.axis_index("s") * _NC + lax.axis_index("c")
    base = wid * 32
    pltpu.sync_copy(idx_hbm.at[pl.ds(base, 32)], idx_v)
    pltpu.async_copy(nodes_hbm.at[idx_v], rows_v, sem).wait()
    pltpu.sync_copy(rows_v, out_hbm.at[pl.ds(base, 32)])


def _tc_body(calls_ref, bnode_ref, we_ref, ws0_ref, wm0_ref, ws1_ref, wm1_ref,
             out_ref, graph_ref):
    f32 = jnp.float32
    A3 = (calls_ref[...] > -1).astype(f32)
    deg2 = jnp.sum(A3, axis=2)
    col2 = jnp.sum(A3, axis=1)
    invdeg3 = (1.0 / jnp.maximum(deg2, 1.0))[..., None]

    node_sum = jnp.dot(bnode_ref[...], we_ref[...], preferred_element_type=f32)
    nodes3 = node_sum.reshape(_M, _S, 128) * invdeg3
    nodes = nodes3.reshape(_N, 128)

    def abmm(h3, d):
        acc = jnp.zeros((_M, _S, d), f32)
        for j in range(_S):
            acc = acc + A3[:, :, j:j + 1] * h3[:, j:j + 1, :]
        return (acc * invdeg3).reshape(_N, d)

    wm0 = wm0_ref[...]
    w0 = ws0_ref[...] + wm0[128:]
    h1 = jnp.maximum(
        jnp.dot(nodes, w0, preferred_element_type=f32)
        + jnp.dot(abmm(nodes3, 128), wm0[:128], preferred_element_type=f32),
        0.0)
    wm1 = wm1_ref[...]
    pre = (jnp.dot(h1, ws1_ref[...], preferred_element_type=f32)
           + jnp.dot(nodes, wm1[256:], preferred_element_type=f32)
           + jnp.dot(abmm(h1.reshape(_M, _S, _DHID), _DHID), wm1[:256],
                     preferred_element_type=f32))
    out = jnp.maximum(pre, 0.0)
    out_ref[...] = out
    mask3 = ((deg2 + col2) > 0.0).astype(f32)[..., None]
    msum = jnp.sum(jnp.sum(out.reshape(_M, _S, _DHID) * mask3, axis=1),
                   axis=0, keepdims=True)
    graph_ref[...] = msum / jnp.sum(mask3)


def kernel(formulas, calls, num_literals, rm_id, state_id, lit_table,
           W_edge, W_self0, W_msg0, W_self1, W_msg1):
    calls3 = calls[..., 0]
    a_bool = calls3 > -1
    nlit = jnp.maximum(num_literals[..., 0], 1)
    lmask = (jnp.arange(_L, dtype=jnp.int32)[None, None, None, :]
             < nlit[..., None]) & a_bool[..., None]
    ids = jnp.where(lmask, formulas[:, :, :, 0, :], _V).astype(jnp.int32)
    ids_t = ids.reshape(_NW, _NPW, _SLOTS)
    table_pk = jax.lax.bitcast_convert_type(
        lit_table.astype(jnp.bfloat16).reshape(_V, _TW, 2), jnp.int32)
    table_pad = jnp.pad(table_pk, ((0, 1), (0, 1))).reshape(-1)

    mesh = plsc.VectorSubcoreMesh(core_axis_name="c", subcore_axis_name="s",
                                  num_cores=_NC)

    bagsum = functools.partial(
        pl.kernel, mesh=mesh,
        compiler_params=pltpu.CompilerParams(needs_layout_passes=False),
        out_type=jax.ShapeDtypeStruct((_N * _DLIT,), jnp.float32),
        scratch_types=[
            pltpu.VMEM((_TROWS * _TSTRIDE,), jnp.int32),
            pltpu.VMEM((_NPW, _SLOTS), jnp.int32),
            pltpu.VMEM((_NPW * _DLIT,), jnp.float32),
        ],
    )(_sc_bagsum_body)
    bnode = bagsum(table_pad, ids_t).reshape(_N, _DLIT)

    out_nodes, graph = pl.pallas_call(
        _tc_body,
        out_shape=(jax.ShapeDtypeStruct((_N, _DHID), jnp.float32),
                   jax.ShapeDtypeStruct((1, _DHID), jnp.float32)),
    )(calls3, bnode, W_edge, W_self0, W_msg0, W_self1, W_msg1)

    idx_b = (rm_id * _S + state_id).astype(jnp.int32)
    sel = functools.partial(
        pl.kernel, mesh=mesh,
        out_type=jax.ShapeDtypeStruct((idx_b.shape[0], _DHID), jnp.float32),
        scratch_types=[
            pltpu.VMEM((32,), jnp.int32),
            pltpu.VMEM((32, _DHID), jnp.float32),
            pltpu.SemaphoreType.DMA,
        ],
    )(_sc_sel_body)(out_nodes, idx_b)

    left = jnp.broadcast_to(graph, (idx_b.shape[0], _DHID))
    return jnp.concatenate([left, sel], axis=1)


# speedup vs baseline: 1.4594x; 1.2765x over previous
import functools

import jax
import jax.numpy as jnp
from jax import lax
from jax.experimental import pallas as pl
from jax.experimental.pallas import tpu as pltpu
from jax.experimental.pallas import tpu_sc as plsc

_M = 256
_S = 16
_L = 8
_V = 1024
_N = _M * _S
_DLIT = 64
_DHID = 256
_NC = 2
_NS = 16
_NW = _NC * _NS
_NPW = _N // _NW
_SLOTS = _S * _L
_TROWS = _V + 1
_TW = _DLIT // 2
_TSTRIDE = _TW + 1


def _sc_bagsum_body(table_hbm, form_hbm, nlit_hbm, out_hbm,
                    table_v, ids_v, nlit_v, stage_v):
    wid = lax.axis_index("s") * _NC + lax.axis_index("c")
    pltpu.sync_copy(table_hbm, table_v)
    pltpu.sync_copy(form_hbm.at[wid], ids_v)
    pltpu.sync_copy(nlit_hbm.at[wid], nlit_v)
    cols = lax.iota(jnp.int32, 16)
    lsub = jnp.bitwise_and(cols, 7)
    lo8 = cols < 8
    zero16 = jnp.zeros((16,), jnp.float32)
    zbf = jnp.zeros((32,), jnp.bfloat16)
    padid = jnp.full((16,), _V, jnp.int32)

    def n_body(n, _):
        nlit16 = nlit_v[n, :]

        def sb_body(sb, faccs):
            raw16 = ids_v[n, pl.ds(sb * 16, 16)]
            t0 = nlit16.at[jnp.full((16,), 2 * sb, jnp.int32)].get(
                mode="promise_in_bounds")
            t1 = nlit16.at[jnp.full((16,), 2 * sb + 1, jnp.int32)].get(
                mode="promise_in_bounds")
            thr = jnp.where(lo8, t0, t1)
            ids16 = jnp.where(lsub < thr, raw16, padid)
            p0, p1 = zbf, zbf
            for j in range(16):
                idj = ids16.at[jnp.full((16,), j, jnp.int32)].get(
                    mode="promise_in_bounds")
                base = idj * _TSTRIDE + cols
                g0 = plsc.load_gather(table_v, [base])
                g1 = plsc.load_gather(table_v, [base + 16])
                p0 = p0 + plsc.bitcast(g0, jnp.bfloat16)
                p1 = p1 + plsc.bitcast(g1, jnp.bfloat16)
            a0, b0 = plsc.unpack(p0, format=plsc.PackFormat.INTERLEAVED)
            a1, b1 = plsc.unpack(p1, format=plsc.PackFormat.INTERLEAVED)
            return (faccs[0] + a0, faccs[1] + b0, faccs[2] + a1, faccs[3] + b1)

        faccs = lax.fori_loop(0, _SLOTS // 16, sb_body, (zero16,) * 4)
        for k in range(4):
            off = (k // 2) * 32 + (k % 2)
            plsc.store_scatter(stage_v, [n * _DLIT + off + 2 * cols], faccs[k])
        return 0

    lax.fori_loop(0, _NPW, n_body, 0)
    pltpu.sync_copy(stage_v, out_hbm.at[pl.ds(wid * _NPW * _DLIT, _NPW * _DLIT)])


def _sc_sel_body(nodes_hbm, idx_hbm, graph_hbm, out_hbm,
                 idx_v, rows_v, graph_v, stage_v, sem):
    wid = lax.axis_index("s") * _NC + lax.axis_index("c")
    base = wid * 32
    pltpu.sync_copy(idx_hbm.at[pl.ds(base, 32)], idx_v)
    pltpu.sync_copy(graph_hbm, graph_v)
    pltpu.async_copy(nodes_hbm.at[idx_v], rows_v, sem).wait()
    gvecs = [graph_v[pl.ds(c * 16, 16)] for c in range(16)]
    for r in range(32):
        for c in range(16):
            stage_v[r, pl.ds(c * 16, 16)] = gvecs[c]
            stage_v[r, pl.ds(_DHID + c * 16, 16)] = rows_v[r, pl.ds(c * 16, 16)]
    pltpu.sync_copy(stage_v, out_hbm.at[pl.ds(base, 32)])


def _tc_body(calls_ref, bnode_ref, we_ref, ws0_ref, wm0_ref, ws1_ref, wm1_ref,
             out_ref, graph_ref):
    f32 = jnp.float32
    A3 = (calls_ref[...] > -1).astype(f32)
    deg2 = jnp.sum(A3, axis=2)
    col2 = jnp.sum(A3, axis=1)
    invdeg3 = (1.0 / jnp.maximum(deg2, 1.0))[..., None]

    node_sum = jnp.dot(bnode_ref[...], we_ref[...], preferred_element_type=f32)
    nodes3 = node_sum.reshape(_M, _S, 128) * invdeg3
    nodes = nodes3.reshape(_N, 128)

    def abmm(h3, d):
        acc = jnp.zeros((_M, _S, d), f32)
        for j in range(_S):
            acc = acc + A3[:, :, j:j + 1] * h3[:, j:j + 1, :]
        return (acc * invdeg3).reshape(_N, d)

    wm0 = wm0_ref[...]
    w0 = ws0_ref[...] + wm0[128:]
    h1 = jnp.maximum(
        jnp.dot(nodes, w0, preferred_element_type=f32)
        + jnp.dot(abmm(nodes3, 128), wm0[:128], preferred_element_type=f32),
        0.0)
    wm1 = wm1_ref[...]
    pre = (jnp.dot(h1, ws1_ref[...], preferred_element_type=f32)
           + jnp.dot(nodes, wm1[256:], preferred_element_type=f32)
           + jnp.dot(abmm(h1.reshape(_M, _S, _DHID), _DHID), wm1[:256],
                     preferred_element_type=f32))
    out = jnp.maximum(pre, 0.0)
    out_ref[...] = out
    mask3 = ((deg2 + col2) > 0.0).astype(f32)[..., None]
    msum = jnp.sum(jnp.sum(out.reshape(_M, _S, _DHID) * mask3, axis=1),
                   axis=0, keepdims=True)
    graph_ref[...] = msum / jnp.sum(mask3)


def kernel(formulas, calls, num_literals, rm_id, state_id, lit_table,
           W_edge, W_self0, W_msg0, W_self1, W_msg1):
    calls3 = calls[..., 0]
    nlit_eff = jnp.where(calls3 > -1,
                         jnp.maximum(num_literals[..., 0], 1),
                         0).astype(jnp.int32).reshape(_NW, _NPW, _S)
    form_r = formulas.reshape(_NW, _NPW, _SLOTS).astype(jnp.int32)
    table_pk = jax.lax.bitcast_convert_type(
        lit_table.astype(jnp.bfloat16).reshape(_V, _TW, 2), jnp.int32)
    table_pad = jnp.pad(table_pk, ((0, 1), (0, 1))).reshape(-1)

    mesh = plsc.VectorSubcoreMesh(core_axis_name="c", subcore_axis_name="s",
                                  num_cores=_NC)

    bagsum = functools.partial(
        pl.kernel, mesh=mesh,
        compiler_params=pltpu.CompilerParams(needs_layout_passes=False),
        out_type=jax.ShapeDtypeStruct((_N * _DLIT,), jnp.float32),
        scratch_types=[
            pltpu.VMEM((_TROWS * _TSTRIDE,), jnp.int32),
            pltpu.VMEM((_NPW, _SLOTS), jnp.int32),
            pltpu.VMEM((_NPW, _S), jnp.int32),
            pltpu.VMEM((_NPW * _DLIT,), jnp.float32),
        ],
    )(_sc_bagsum_body)
    bnode = bagsum(table_pad, form_r, nlit_eff).reshape(_N, _DLIT)

    out_nodes, graph = pl.pallas_call(
        _tc_body,
        out_shape=(jax.ShapeDtypeStruct((_N, _DHID), jnp.float32),
                   jax.ShapeDtypeStruct((1, _DHID), jnp.float32)),
    )(calls3, bnode, W_edge, W_self0, W_msg0, W_self1, W_msg1)

    idx_b = (rm_id * _S + state_id).astype(jnp.int32)
    B = idx_b.shape[0]
    return functools.partial(
        pl.kernel, mesh=mesh,
        out_type=jax.ShapeDtypeStruct((B, 2 * _DHID), jnp.float32),
        scratch_types=[
            pltpu.VMEM((32,), jnp.int32),
            pltpu.VMEM((32, _DHID), jnp.float32),
            pltpu.VMEM((_DHID,), jnp.float32),
            pltpu.VMEM((32, 2 * _DHID), jnp.float32),
            pltpu.SemaphoreType.DMA,
        ],
    )(_sc_sel_body)(out_nodes, idx_b, graph.reshape(_DHID))


# speedup vs baseline: 1.6326x; 1.1186x over previous
import functools

import jax
import jax.numpy as jnp
from jax import lax
from jax.experimental import pallas as pl
from jax.experimental.pallas import tpu as pltpu
from jax.experimental.pallas import tpu_sc as plsc

_M = 256
_S = 16
_L = 8
_V = 1024
_N = _M * _S
_DLIT = 64
_DHID = 256
_NC = 2
_NS = 16
_NW = _NC * _NS
_NPW = _N // _NW
_SLOTS = _S * _L
_TROWS = _V + 1
_TW = _DLIT // 2
_TSTRIDE = _TW + 1


def _sc_bagsum_body(table_hbm, form_hbm, nlit_hbm, out_hbm,
                    table_v, ids_v, nlit_v, stage_v):
    wid = lax.axis_index("s") * _NC + lax.axis_index("c")
    pltpu.sync_copy(table_hbm, table_v)
    pltpu.sync_copy(form_hbm.at[wid], ids_v)
    pltpu.sync_copy(nlit_hbm.at[wid], nlit_v)
    cols = lax.iota(jnp.int32, 16)
    lsub = jnp.bitwise_and(cols, 7)
    lo8 = cols < 8
    zero16 = jnp.zeros((16,), jnp.float32)
    zbf = jnp.zeros((32,), jnp.bfloat16)
    padid = jnp.full((16,), _V, jnp.int32)

    def n_body(n, _):
        nlit16 = nlit_v[n, :]

        def sb_body(sb, faccs):
            raw16 = ids_v[n, pl.ds(sb * 16, 16)]
            t0 = nlit16.at[jnp.full((16,), 2 * sb, jnp.int32)].get(
                mode="promise_in_bounds")
            t1 = nlit16.at[jnp.full((16,), 2 * sb + 1, jnp.int32)].get(
                mode="promise_in_bounds")
            thr = jnp.where(lo8, t0, t1)
            ids16 = jnp.where(lsub < thr, raw16, padid)
            p0, p1 = zbf, zbf
            for j in range(16):
                idj = ids16.at[jnp.full((16,), j, jnp.int32)].get(
                    mode="promise_in_bounds")
                base = idj * _TSTRIDE + cols
                g0 = plsc.load_gather(table_v, [base])
                g1 = plsc.load_gather(table_v, [base + 16])
                p0 = p0 + plsc.bitcast(g0, jnp.bfloat16)
                p1 = p1 + plsc.bitcast(g1, jnp.bfloat16)
            a0, b0 = plsc.unpack(p0, format=plsc.PackFormat.INTERLEAVED)
            a1, b1 = plsc.unpack(p1, format=plsc.PackFormat.INTERLEAVED)
            return (faccs[0] + a0, faccs[1] + b0, faccs[2] + a1, faccs[3] + b1)

        faccs = lax.fori_loop(0, _SLOTS // 16, sb_body, (zero16,) * 4)
        for k in range(4):
            off = (k // 2) * 32 + (k % 2)
            plsc.store_scatter(stage_v, [n * _DLIT + off + 2 * cols], faccs[k])
        return 0

    lax.fori_loop(0, _NPW, n_body, 0)
    pltpu.sync_copy(stage_v, out_hbm.at[pl.ds(wid * _NPW * _DLIT, _NPW * _DLIT)])


def _sc_sel_body(nodes_hbm, idx_hbm, graph_hbm, out_hbm,
                 idx_v, rows_v, graph_v, stage_v, sem):
    wid = lax.axis_index("s") * _NC + lax.axis_index("c")
    base = wid * 32
    pltpu.sync_copy(idx_hbm.at[pl.ds(base, 32)], idx_v)
    pltpu.sync_copy(graph_hbm, graph_v)
    pltpu.async_copy(nodes_hbm.at[idx_v], rows_v, sem).wait()
    gvecs = [graph_v[pl.ds(c * 16, 16)] for c in range(16)]
    for r in range(32):
        for c in range(16):
            stage_v[r, pl.ds(c * 16, 16)] = gvecs[c]
            stage_v[r, pl.ds(_DHID + c * 16, 16)] = rows_v[r, pl.ds(c * 16, 16)]
    pltpu.sync_copy(stage_v, out_hbm.at[pl.ds(base, 32)])


def _tc_body(calls_ref, bnode_ref, we_ref, ws0_ref, wm0_ref, ws1_ref, wm1_ref,
             out_ref, graph_ref):
    f32 = jnp.float32
    A3 = (calls_ref[...] > -1).astype(f32)
    deg2 = jnp.sum(A3, axis=2)
    col2 = jnp.sum(A3, axis=1)
    invdeg3 = (1.0 / jnp.maximum(deg2, 1.0))[..., None]

    node_sum = jnp.dot(bnode_ref[...], we_ref[...], preferred_element_type=f32)
    nodes3 = node_sum.reshape(_M, _S, 128) * invdeg3
    nodes = nodes3.reshape(_N, 128)

    def abmm(h3, d):
        acc = lax.dot_general(A3, h3, (((2,), (1,)), ((0,), (0,))),
                              preferred_element_type=f32)
        return (acc * invdeg3).reshape(_N, d)

    wm0 = wm0_ref[...]
    w0 = ws0_ref[...] + wm0[128:]
    h1 = jnp.maximum(
        jnp.dot(nodes, w0, preferred_element_type=f32)
        + jnp.dot(abmm(nodes3, 128), wm0[:128], preferred_element_type=f32),
        0.0)
    wm1 = wm1_ref[...]
    pre = (jnp.dot(h1, ws1_ref[...], preferred_element_type=f32)
           + jnp.dot(nodes, wm1[256:], preferred_element_type=f32)
           + jnp.dot(abmm(h1.reshape(_M, _S, _DHID), _DHID), wm1[:256],
                     preferred_element_type=f32))
    out = jnp.maximum(pre, 0.0)
    out_ref[...] = out
    mask3 = ((deg2 + col2) > 0.0).astype(f32)[..., None]
    msum = jnp.sum(jnp.sum(out.reshape(_M, _S, _DHID) * mask3, axis=1),
                   axis=0, keepdims=True)
    graph_ref[...] = msum / jnp.sum(mask3)


def kernel(formulas, calls, num_literals, rm_id, state_id, lit_table,
           W_edge, W_self0, W_msg0, W_self1, W_msg1):
    calls3 = calls[..., 0]
    nlit_eff = jnp.where(calls3 > -1,
                         jnp.maximum(num_literals[..., 0], 1),
                         0).astype(jnp.int32).reshape(_NW, _NPW, _S)
    form_r = formulas.reshape(_NW, _NPW, _SLOTS).astype(jnp.int32)
    table_pk = jax.lax.bitcast_convert_type(
        lit_table.astype(jnp.bfloat16).reshape(_V, _TW, 2), jnp.int32)
    table_pad = jnp.pad(table_pk, ((0, 1), (0, 1))).reshape(-1)

    mesh = plsc.VectorSubcoreMesh(core_axis_name="c", subcore_axis_name="s",
                                  num_cores=_NC)

    bagsum = functools.partial(
        pl.kernel, mesh=mesh,
        compiler_params=pltpu.CompilerParams(needs_layout_passes=False),
        out_type=jax.ShapeDtypeStruct((_N * _DLIT,), jnp.float32),
        scratch_types=[
            pltpu.VMEM((_TROWS * _TSTRIDE,), jnp.int32),
            pltpu.VMEM((_NPW, _SLOTS), jnp.int32),
            pltpu.VMEM((_NPW, _S), jnp.int32),
            pltpu.VMEM((_NPW * _DLIT,), jnp.float32),
        ],
    )(_sc_bagsum_body)
    bnode = bagsum(table_pad, form_r, nlit_eff).reshape(_N, _DLIT)

    out_nodes, graph = pl.pallas_call(
        _tc_body,
        out_shape=(jax.ShapeDtypeStruct((_N, _DHID), jnp.float32),
                   jax.ShapeDtypeStruct((1, _DHID), jnp.float32)),
    )(calls3, bnode, W_edge, W_self0, W_msg0, W_self1, W_msg1)

    idx_b = (rm_id * _S + state_id).astype(jnp.int32)
    B = idx_b.shape[0]
    return functools.partial(
        pl.kernel, mesh=mesh,
        out_type=jax.ShapeDtypeStruct((B, 2 * _DHID), jnp.float32),
        scratch_types=[
            pltpu.VMEM((32,), jnp.int32),
            pltpu.VMEM((32, _DHID), jnp.float32),
            pltpu.VMEM((_DHID,), jnp.float32),
            pltpu.VMEM((32, 2 * _DHID), jnp.float32),
            pltpu.SemaphoreType.DMA,
        ],
    )(_sc_sel_body)(out_nodes, idx_b, graph.reshape(_DHID))


# speedup vs baseline: 1.6533x; 1.0127x over previous
import functools

import jax
import jax.numpy as jnp
from jax import lax
from jax.experimental import pallas as pl
from jax.experimental.pallas import tpu as pltpu
from jax.experimental.pallas import tpu_sc as plsc

_M = 256
_S = 16
_L = 8
_V = 1024
_N = _M * _S
_DLIT = 64
_DHID = 256
_NC = 2
_NS = 16
_NW = _NC * _NS
_NPW = _N // _NW
_SLOTS = _S * _L
_TROWS = _V + 1
_TW = _DLIT // 2
_TSTRIDE = _TW + 1


def _sc_bagsum_body(table_hbm, form_hbm, nlit_hbm, out_hbm,
                    table_v, ids_v, nlit_v, stage_v):
    wid = lax.axis_index("s") * _NC + lax.axis_index("c")
    pltpu.sync_copy(table_hbm, table_v)
    pltpu.sync_copy(form_hbm.at[wid], ids_v)
    pltpu.sync_copy(nlit_hbm.at[wid], nlit_v)
    cols = lax.iota(jnp.int32, 16)
    lsub = jnp.bitwise_and(cols, 7)
    lo8 = cols < 8
    zero16 = jnp.zeros((16,), jnp.float32)
    zbf = jnp.zeros((32,), jnp.bfloat16)
    padid = jnp.full((16,), _V, jnp.int32)

    def n_body(n, _):
        nlit16 = nlit_v[n, :]

        def sb_body(sb, faccs):
            raw16 = ids_v[n, pl.ds(sb * 16, 16)]
            t0 = nlit16.at[jnp.full((16,), 2 * sb, jnp.int32)].get(
                mode="promise_in_bounds")
            t1 = nlit16.at[jnp.full((16,), 2 * sb + 1, jnp.int32)].get(
                mode="promise_in_bounds")
            thr = jnp.where(lo8, t0, t1)
            ids16 = jnp.where(lsub < thr, raw16, padid)
            p0, p1 = zbf, zbf
            for j in range(16):
                idj = ids16.at[jnp.full((16,), j, jnp.int32)].get(
                    mode="promise_in_bounds")
                base = idj * _TSTRIDE + cols
                g0 = plsc.load_gather(table_v, [base])
                g1 = plsc.load_gather(table_v, [base + 16])
                p0 = p0 + plsc.bitcast(g0, jnp.bfloat16)
                p1 = p1 + plsc.bitcast(g1, jnp.bfloat16)
            a0, b0 = plsc.unpack(p0, format=plsc.PackFormat.INTERLEAVED)
            a1, b1 = plsc.unpack(p1, format=plsc.PackFormat.INTERLEAVED)
            return (faccs[0] + a0, faccs[1] + b0, faccs[2] + a1, faccs[3] + b1)

        faccs = lax.fori_loop(0, _SLOTS // 16, sb_body, (zero16,) * 4)
        for k in range(4):
            off = (k // 2) * 32 + (k % 2)
            plsc.store_scatter(stage_v, [n * _DLIT + off + 2 * cols], faccs[k])
        return 0

    lax.fori_loop(0, _NPW, n_body, 0)
    pltpu.sync_copy(stage_v, out_hbm.at[pl.ds(wid * _NPW * _DLIT, _NPW * _DLIT)])


def _sc_sel_body(nodes_hbm, idx_hbm, graph_hbm, out_hbm,
                 idx_v, rows_v, graph_v, stage_v, sem):
    wid = lax.axis_index("s") * _NC + lax.axis_index("c")
    base = wid * 32
    pltpu.sync_copy(idx_hbm.at[pl.ds(base, 32)], idx_v)
    pltpu.sync_copy(graph_hbm, graph_v)
    pltpu.async_copy(nodes_hbm.at[idx_v], rows_v, sem).wait()
    gvecs = [graph_v[pl.ds(c * 16, 16)] for c in range(16)]
    for r in range(32):
        for c in range(16):
            stage_v[r, pl.ds(c * 16, 16)] = gvecs[c]
            stage_v[r, pl.ds(_DHID + c * 16, 16)] = rows_v[r, pl.ds(c * 16, 16)]
    pltpu.sync_copy(stage_v, out_hbm.at[pl.ds(base, 32)])


def _tc_body(calls_ref, bnode_ref, idx_ref, we_ref, ws0_ref, wm0_ref, ws1_ref,
             wm1_ref, out_ref):
    f32 = jnp.float32
    A3 = (calls_ref[...] > -1).astype(f32)
    deg2 = jnp.sum(A3, axis=2)
    col2 = jnp.sum(A3, axis=1)
    invdeg3 = (1.0 / jnp.maximum(deg2, 1.0))[..., None]

    node_sum = jnp.dot(bnode_ref[...], we_ref[...], preferred_element_type=f32)
    nodes3 = node_sum.reshape(_M, _S, 128) * invdeg3
    nodes = nodes3.reshape(_N, 128)

    def abmm(h3, d):
        acc = lax.dot_general(A3, h3, (((2,), (1,)), ((0,), (0,))),
                              preferred_element_type=f32)
        return (acc * invdeg3).reshape(_N, d)

    wm0 = wm0_ref[...]
    w0 = ws0_ref[...] + wm0[128:]
    h1 = jnp.maximum(
        jnp.dot(nodes, w0, preferred_element_type=f32)
        + jnp.dot(abmm(nodes3, 128), wm0[:128], preferred_element_type=f32),
        0.0)
    wm1 = wm1_ref[...]
    pre = (jnp.dot(h1, ws1_ref[...], preferred_element_type=f32)
           + jnp.dot(nodes, wm1[256:], preferred_element_type=f32)
           + jnp.dot(abmm(h1.reshape(_M, _S, _DHID), _DHID), wm1[:256],
                     preferred_element_type=f32))
    out = jnp.maximum(pre, 0.0)
    mask3 = ((deg2 + col2) > 0.0).astype(f32)[..., None]
    msum = jnp.sum(jnp.sum(out.reshape(_M, _S, _DHID) * mask3, axis=1),
                   axis=0, keepdims=True)
    graph = msum / jnp.sum(mask3)
    b = idx_ref.shape[0]
    onehot = (idx_ref[...] ==
              lax.broadcasted_iota(jnp.int32, (b, _N), 1)).astype(f32)
    sel = jnp.dot(onehot, out, preferred_element_type=f32)
    out_ref[:, pl.ds(0, _DHID)] = jnp.broadcast_to(graph, (b, _DHID))
    out_ref[:, pl.ds(_DHID, _DHID)] = sel


def kernel(formulas, calls, num_literals, rm_id, state_id, lit_table,
           W_edge, W_self0, W_msg0, W_self1, W_msg1):
    calls3 = calls[..., 0]
    nlit_eff = jnp.where(calls3 > -1,
                         jnp.maximum(num_literals[..., 0], 1),
                         0).astype(jnp.int32).reshape(_NW, _NPW, _S)
    form_r = formulas.reshape(_NW, _NPW, _SLOTS).astype(jnp.int32)
    table_pk = jax.lax.bitcast_convert_type(
        lit_table.astype(jnp.bfloat16).reshape(_V, _TW, 2), jnp.int32)
    table_pad = jnp.pad(table_pk, ((0, 1), (0, 1))).reshape(-1)

    mesh = plsc.VectorSubcoreMesh(core_axis_name="c", subcore_axis_name="s",
                                  num_cores=_NC)

    bagsum = functools.partial(
        pl.kernel, mesh=mesh,
        compiler_params=pltpu.CompilerParams(needs_layout_passes=False),
        out_type=jax.ShapeDtypeStruct((_N * _DLIT,), jnp.float32),
        scratch_types=[
            pltpu.VMEM((_TROWS * _TSTRIDE,), jnp.int32),
            pltpu.VMEM((_NPW, _SLOTS), jnp.int32),
            pltpu.VMEM((_NPW, _S), jnp.int32),
            pltpu.VMEM((_NPW * _DLIT,), jnp.float32),
        ],
    )(_sc_bagsum_body)
    bnode = bagsum(table_pad, form_r, nlit_eff).reshape(_N, _DLIT)

    idx_b = (rm_id * _S + state_id).astype(jnp.int32)
    B = idx_b.shape[0]
    return pl.pallas_call(
        _tc_body,
        out_shape=jax.ShapeDtypeStruct((B, 2 * _DHID), jnp.float32),
    )(calls3, bnode, idx_b[:, None], W_edge, W_self0, W_msg0, W_self1, W_msg1)


# speedup vs baseline: 1.6551x; 1.0011x over previous
import functools

import jax
import jax.numpy as jnp
from jax import lax
from jax.experimental import pallas as pl
from jax.experimental.pallas import tpu as pltpu
from jax.experimental.pallas import tpu_sc as plsc

_M = 256
_S = 16
_L = 8
_V = 1024
_N = _M * _S
_DLIT = 64
_DHID = 256
_NC = 2
_NS = 16
_NW = _NC * _NS
_NPW = _N // _NW
_SLOTS = _S * _L
_TROWS = _V + 1
_TW = _DLIT // 2
_TSTRIDE = _TW + 1


def _sc_bagsum_body(table_hbm, form_hbm, nlit_hbm, out_hbm,
                    table_v, ids_v, nlit_v, stage_v):
    wid = lax.axis_index("s") * _NC + lax.axis_index("c")
    pltpu.sync_copy(table_hbm, table_v)
    pltpu.sync_copy(form_hbm.at[wid], ids_v)
    pltpu.sync_copy(nlit_hbm.at[wid], nlit_v)
    cols = lax.iota(jnp.int32, 16)
    lsub = jnp.bitwise_and(cols, 7)
    lo8 = cols < 8
    zero16 = jnp.zeros((16,), jnp.float32)
    zbf = jnp.zeros((32,), jnp.bfloat16)
    padid = jnp.full((16,), _V, jnp.int32)

    def n_body(n, _):
        nlit16 = nlit_v[n, :]

        def sb_body(sb, faccs):
            raw16 = ids_v[n, pl.ds(sb * 16, 16)]
            t0 = nlit16.at[jnp.full((16,), 2 * sb, jnp.int32)].get(
                mode="promise_in_bounds")
            t1 = nlit16.at[jnp.full((16,), 2 * sb + 1, jnp.int32)].get(
                mode="promise_in_bounds")
            thr = jnp.where(lo8, t0, t1)
            ids16 = jnp.where(lsub < thr, raw16, padid)
            p0, p1 = zbf, zbf
            for j in range(16):
                idj = ids16.at[jnp.full((16,), j, jnp.int32)].get(
                    mode="promise_in_bounds")
                base = idj * _TSTRIDE + cols
                g0 = plsc.load_gather(table_v, [base])
                g1 = plsc.load_gather(table_v, [base + 16])
                p0 = p0 + plsc.bitcast(g0, jnp.bfloat16)
                p1 = p1 + plsc.bitcast(g1, jnp.bfloat16)
            a0, b0 = plsc.unpack(p0, format=plsc.PackFormat.INTERLEAVED)
            a1, b1 = plsc.unpack(p1, format=plsc.PackFormat.INTERLEAVED)
            return (faccs[0] + a0, faccs[1] + b0, faccs[2] + a1, faccs[3] + b1)

        faccs = lax.fori_loop(0, _SLOTS // 16, sb_body, (zero16,) * 4)
        for k in range(4):
            off = (k // 2) * 32 + (k % 2)
            plsc.store_scatter(stage_v, [n * _DLIT + off + 2 * cols], faccs[k])
        return 0

    lax.fori_loop(0, _NPW, n_body, 0)
    pltpu.sync_copy(stage_v, out_hbm.at[pl.ds(wid * _NPW * _DLIT, _NPW * _DLIT)])


def _tc_body(calls_ref, bnode_ref, idx_ref, we_ref, ws0_ref, wm0_ref, ws1_ref,
             wm1_ref, out_ref):
    f32 = jnp.float32
    A3 = (calls_ref[...] > -1).astype(f32)
    deg2 = jnp.sum(A3, axis=2)
    col2 = jnp.sum(A3, axis=1)
    invdeg3 = (1.0 / jnp.maximum(deg2, 1.0))[..., None]

    node_sum = jnp.dot(bnode_ref[...], we_ref[...], preferred_element_type=f32)
    nodes3 = node_sum.reshape(_M, _S, 128) * invdeg3
    nodes = nodes3.reshape(_N, 128)

    def abmm(h3, d):
        acc = lax.dot_general(A3, h3, (((2,), (1,)), ((0,), (0,))),
                              preferred_element_type=f32)
        return (acc * invdeg3).reshape(_N, d)

    wm0 = wm0_ref[...]
    w0 = ws0_ref[...] + wm0[128:]
    h1 = jnp.maximum(
        jnp.dot(nodes, w0, preferred_element_type=f32)
        + jnp.dot(abmm(nodes3, 128), wm0[:128], preferred_element_type=f32),
        0.0)
    wm1 = wm1_ref[...]
    pre = (jnp.dot(h1, ws1_ref[...], preferred_element_type=f32)
           + jnp.dot(nodes, wm1[256:], preferred_element_type=f32)
           + jnp.dot(abmm(h1.reshape(_M, _S, _DHID), _DHID), wm1[:256],
                     preferred_element_type=f32))
    out = jnp.maximum(pre, 0.0)
    mask3 = ((deg2 + col2) > 0.0).astype(f32)[..., None]
    msum = jnp.sum(jnp.sum(out.reshape(_M, _S, _DHID) * mask3, axis=1),
                   axis=0, keepdims=True)
    graph = msum / jnp.sum(mask3)
    b = idx_ref.shape[0]
    onehot = (idx_ref[...] ==
              lax.broadcasted_iota(jnp.int32, (b, _N), 1)).astype(f32)
    sel = jnp.dot(onehot, out, preferred_element_type=f32)
    out_ref[:, pl.ds(0, _DHID)] = jnp.broadcast_to(graph, (b, _DHID))
    out_ref[:, pl.ds(_DHID, _DHID)] = sel


def kernel(formulas, calls, num_literals, rm_id, state_id, lit_table,
           W_edge, W_self0, W_msg0, W_self1, W_msg1):
    calls3 = calls[..., 0]
    nlit_eff = jnp.where(calls3 > -1,
                         jnp.maximum(num_literals[..., 0], 1),
                         0).astype(jnp.int32).reshape(_NW, _NPW, _S)
    form_r = formulas.reshape(_NW, _NPW, _SLOTS).astype(jnp.int32)
    table_pk = jax.lax.bitcast_convert_type(
        lit_table.astype(jnp.bfloat16).reshape(_V, _TW, 2), jnp.int32)
    table_pad = jnp.pad(table_pk, ((0, 1), (0, 1))).reshape(-1)

    mesh = plsc.VectorSubcoreMesh(core_axis_name="c", subcore_axis_name="s",
                                  num_cores=_NC)

    bagsum = functools.partial(
        pl.kernel, mesh=mesh,
        compiler_params=pltpu.CompilerParams(needs_layout_passes=False),
        out_type=jax.ShapeDtypeStruct((_N * _DLIT,), jnp.float32),
        scratch_types=[
            pltpu.VMEM((_TROWS * _TSTRIDE,), jnp.int32),
            pltpu.VMEM((_NPW, _SLOTS), jnp.int32),
            pltpu.VMEM((_NPW, _S), jnp.int32),
            pltpu.VMEM((_NPW * _DLIT,), jnp.float32),
        ],
    )(_sc_bagsum_body)
    bnode = bagsum(table_pad, form_r, nlit_eff).reshape(_N, _DLIT)

    idx_b = (rm_id * _S + state_id).astype(jnp.int32)
    B = idx_b.shape[0]
    return pl.pallas_call(
        _tc_body,
        out_shape=jax.ShapeDtypeStruct((B, 2 * _DHID), jnp.float32),
    )(calls3, bnode, idx_b[:, None], W_edge, W_self0, W_msg0, W_self1, W_msg1)
